# baseline (speedup 1.0000x reference)
import functools

import jax
import jax.numpy as jnp
from jax.experimental import pallas as pl
from jax.experimental.pallas import tpu as pltpu

K = 32
B = 64
N = 32768
D = 64
IPB = 16384
S = (B * N) // IPB
HB = IPB // 2


def _score_kernel(x_ref, a_ref, bmat_ref, o_ref):
    x = x_ref[...]
    proj = jnp.dot(x, bmat_ref[...], preferred_element_type=jnp.float32)
    y = proj * x
    row = jax.lax.broadcasted_iota(jnp.int32, (2, 2 * D), 0)
    lane = jax.lax.broadcasted_iota(jnp.int32, (2, 2 * D), 1)
    halves = (lane // D == row).astype(jnp.float32)
    pred = jax.lax.dot_general(halves, y, (((1,), (1,)), ((), ())),
                               preferred_element_type=jnp.float32)
    o_ref[0] = a_ref[0] * pred


def _topk_kernel(s_ref, o_ref):
    def step(i, carry):
        total, consumed = carry
        s = s_ref[...]
        m = jnp.max(s, axis=1, keepdims=True)
        eq = (s == m)
        cnt = jnp.sum(eq.astype(jnp.float32), axis=1, keepdims=True)
        take = jnp.clip(jnp.float32(K) - consumed, 0.0, cnt)
        total = total + jnp.where(take > 0.0, m, 0.0) * take
        consumed = consumed + take
        s_ref[...] = jnp.where(eq, -jnp.inf, s)
        return total, consumed

    z = jnp.zeros((B, 1), jnp.float32)
    total, _ = jax.lax.fori_loop(0, K, step, (z, z))
    o_ref[...] = total * (1.0 / K)


@functools.partial(jax.jit, static_argnames=())
def kernel(ins_emb, attn, W):
    d = W.shape[0]
    bmat = (jnp.zeros((2 * D, 2 * D), jnp.float32)
            .at[:d, d:D].set(W.T)
            .at[D:D + d, D + d:].set(W.T))
    x2 = ins_emb.reshape(B * N // 2, 2 * D)
    attn_t = attn.reshape(S, HB, 2).transpose(0, 2, 1)

    scores = pl.pallas_call(
        _score_kernel,
        grid=(S,),
        in_specs=[
            pl.BlockSpec((HB, 2 * D), lambda s: (s, 0)),
            pl.BlockSpec((1, 2, HB), lambda s: (s, 0, 0)),
            pl.BlockSpec((2 * D, 2 * D), lambda s: (0, 0)),
        ],
        out_specs=pl.BlockSpec((1, 2, HB), lambda s: (s, 0, 0)),
        out_shape=jax.ShapeDtypeStruct((S, 2, HB), jnp.float32),
    )(x2, attn_t, bmat)
    scores = scores.reshape(B, N)

    out = pl.pallas_call(
        _topk_kernel,
        grid=(1,),
        in_specs=[pl.BlockSpec((B, N), lambda i: (0, 0))],
        out_specs=pl.BlockSpec((B, 1), lambda i: (0, 0)),
        out_shape=jax.ShapeDtypeStruct((B, 1), jnp.float32),
    )(scores)
    return out


# speedup vs baseline: 1.0007x; 1.0007x over previous
import functools

import jax
import jax.numpy as jnp
from jax import lax
from jax.experimental import pallas as pl
from jax.experimental.pallas import tpu as pltpu
from jax.experimental.pallas import tpu_sc as plsc

K = 32
B = 64
N = 32768
D = 64
IPB = 16384
S = (B * N) // IPB
HB = IPB // 2


def _score_kernel(x_ref, a_ref, bmat_ref, o_ref):
    x = x_ref[...]
    proj = jnp.dot(x, bmat_ref[...], preferred_element_type=jnp.float32)
    y = proj * x
    row = jax.lax.broadcasted_iota(jnp.int32, (2, 2 * D), 0)
    lane = jax.lax.broadcasted_iota(jnp.int32, (2, 2 * D), 1)
    halves = (lane // D == row).astype(jnp.float32)
    pred = jax.lax.dot_general(halves, y, (((1,), (1,)), ((), ())),
                               preferred_element_type=jnp.float32)
    o_ref[0] = a_ref[0] * pred


def _topk_kernel(s_ref, o_ref):
    def step(i, carry):
        total, consumed = carry
        s = s_ref[...]
        m = jnp.max(s, axis=1, keepdims=True)
        eq = (s == m)
        cnt = jnp.sum(eq.astype(jnp.float32), axis=1, keepdims=True)
        take = jnp.clip(jnp.float32(K) - consumed, 0.0, cnt)
        total = total + jnp.where(take > 0.0, m, 0.0) * take
        consumed = consumed + take
        s_ref[...] = jnp.where(eq, -jnp.inf, s)
        return total, consumed

    z = jnp.zeros((B, 1), jnp.float32)
    total, _ = jax.lax.fori_loop(0, K, step, (z, z))
    o_ref[...] = total * (1.0 / K)


NG = 128
GCH = 16
NEG = float("-inf")
SC_STAGE = 99


def _sc_topk_body(s_hbm, o_hbm, V, GM, ACC, CAND, CNT, OUTROW):
    wid = lax.axis_index("s") * 2 + lax.axis_index("c")

    def do_row(j, _):
        r = wid * 2 + j
        pltpu.sync_copy(s_hbm.at[r], V)
        if SC_STAGE < 2:
            OUTROW[...] = V[pl.ds(0, 16)]
            pltpu.sync_copy(OUTROW, o_hbm.at[r])
            return 0

        def p1(g, _):
            base = g * (GCH * 16)
            acc = V[pl.ds(base, 16)]
            for t in range(1, GCH):
                acc = jnp.maximum(acc, V[pl.ds(base + t * 16, 16)])
            GM[pl.ds(g * 16, 16)] = acc
            return 0
        lax.fori_loop(0, NG, p1, 0)
        if SC_STAGE < 3:
            OUTROW[...] = GM[pl.ds(0, 16)]
            pltpu.sync_copy(OUTROW, o_hbm.at[r])
            return 0

        for a in range(8):
            def p2(i, m, a=a):
                return jnp.maximum(m, GM[pl.ds((i * 8 + a) * 16, 16)])
            ACC[pl.ds(a * 16, 16)] = lax.fori_loop(
                0, NG // 8, p2, jnp.full((16,), NEG, jnp.float32))
        if SC_STAGE < 4:
            OUTROW[...] = ACC[pl.ds(0, 16)]
            pltpu.sync_copy(OUTROW, o_hbm.at[r])
            return 0

        def ext(k, tv):
            m = jnp.full((16,), NEG, jnp.float32)
            for a in range(8):
                m = jnp.maximum(m, ACC[pl.ds(a * 16, 16)])
            msv = jnp.full((16,), jnp.max(m))
            for a in range(8):
                v = ACC[pl.ds(a * 16, 16)]
                ACC[pl.ds(a * 16, 16)] = jnp.where(v == msv, NEG, v)
            return msv
        tv = lax.fori_loop(0, K, ext, jnp.full((16,), NEG, jnp.float32))
        if SC_STAGE < 5:
            OUTROW[...] = tv
            pltpu.sync_copy(OUTROW, o_hbm.at[r])
            return 0

        def p3a(g, _):
            gm = GM[pl.ds(g * 16, 16)]
            CNT[pl.ds(g * 16, 16)] = plsc.all_reduce_population_count(gm >= tv)
            return 0
        lax.fori_loop(0, NG, p3a, 0)
        if SC_STAGE < 6:
            OUTROW[...] = CNT[pl.ds(0, 16)].astype(jnp.float32)
            pltpu.sync_copy(OUTROW, o_hbm.at[r])
            return 0

        def p3b(g, ptr):
            def collect(p):
                for t in range(GCH):
                    c = V[pl.ds(g * (GCH * 16) + t * 16, 16)]
                    msk = c >= tv
                    plsc.store_compressed(CAND.at[pl.ds(p, 16)], c, mask=msk)
                    p = p + jnp.max(plsc.all_reduce_population_count(msk))
                return p
            trig = jnp.max(CNT[pl.ds(g * 16, 16)])
            return lax.cond(trig > 0, collect, lambda p: p, ptr)
        ptr = lax.fori_loop(0, NG, p3b, jnp.int32(0))
        if SC_STAGE < 7:
            OUTROW[...] = jnp.full((16,), ptr).astype(jnp.float32)
            pltpu.sync_copy(OUTROW, o_hbm.at[r])
            return 0

        CAND[pl.ds(ptr, 16)] = jnp.full((16,), NEG, jnp.float32)
        nv = lax.shift_right_logical(ptr + 15, 4)

        def rnd(k, carry):
            total, consumed = carry
            def fmax(i, m):
                return jnp.maximum(m, CAND[pl.ds(i * 16, 16)])
            m = lax.fori_loop(0, nv, fmax, jnp.full((16,), NEG, jnp.float32))
            msv = jnp.full((16,), jnp.max(m))

            def cm(i, pc):
                c = CAND[pl.ds(i * 16, 16)]
                eqm = c == msv
                pc = pc + plsc.all_reduce_population_count(eqm)
                CAND[pl.ds(i * 16, 16)] = jnp.where(eqm, NEG, c)
                return pc
            pc = lax.fori_loop(0, nv, cm, jnp.zeros((16,), jnp.int32))
            cntf = pc.astype(jnp.float32)
            take = jnp.clip(jnp.float32(K) - consumed, 0.0, cntf)
            total = total + jnp.where(take > 0.0, msv, 0.0) * take
            return total, consumed + take

        z = jnp.zeros((16,), jnp.float32)
        total, _ = lax.fori_loop(0, K, rnd, (z, z))
        OUTROW[...] = total * (1.0 / K)
        pltpu.sync_copy(OUTROW, o_hbm.at[r])
        return 0

    lax.fori_loop(0, 2, do_row, 0)


def _sc_topk(scores):
    f = pl.kernel(
        _sc_topk_body,
        out_type=jax.ShapeDtypeStruct((B, 16), jnp.float32),
        mesh=plsc.VectorSubcoreMesh(core_axis_name="c", subcore_axis_name="s"),
        scratch_types=[
            pltpu.VMEM((N,), jnp.float32),
            pltpu.VMEM((NG * 16,), jnp.float32),
            pltpu.VMEM((8 * 16,), jnp.float32),
            pltpu.VMEM((N + 16,), jnp.float32),
            pltpu.VMEM((NG * 16,), jnp.int32),
            pltpu.VMEM((16,), jnp.float32),
        ],
    )
    return f(scores)


@functools.partial(jax.jit, static_argnames=())
def kernel(ins_emb, attn, W):
    d = W.shape[0]
    bmat = (jnp.zeros((2 * D, 2 * D), jnp.float32)
            .at[:d, d:D].set(W.T)
            .at[D:D + d, D + d:].set(W.T))
    x2 = ins_emb.reshape(B * N // 2, 2 * D)
    attn_t = attn.reshape(S, HB, 2).transpose(0, 2, 1)

    scores = pl.pallas_call(
        _score_kernel,
        grid=(S,),
        in_specs=[
            pl.BlockSpec((HB, 2 * D), lambda s: (s, 0)),
            pl.BlockSpec((1, 2, HB), lambda s: (s, 0, 0)),
            pl.BlockSpec((2 * D, 2 * D), lambda s: (0, 0)),
        ],
        out_specs=pl.BlockSpec((1, 2, HB), lambda s: (s, 0, 0)),
        out_shape=jax.ShapeDtypeStruct((S, 2, HB), jnp.float32),
    )(x2, attn_t, bmat)
    scores = scores.reshape(B, N)

    out = pl.pallas_call(
        _topk_kernel,
        grid=(1,),
        in_specs=[pl.BlockSpec((B, N), lambda i: (0, 0))],
        out_specs=pl.BlockSpec((B, 1), lambda i: (0, 0)),
        out_shape=jax.ShapeDtypeStruct((B, 1), jnp.float32),
    )(scores)
    return out
